# Initial kernel scaffold; baseline (speedup 1.0000x reference)
import jax
import jax.numpy as jnp
from jax.experimental import pallas as pl


def kernel(x, edge_index, edge_attr, params):
    raise NotImplementedError("write your pallas kernel here")


# speedup vs baseline: 2.5355x; 2.5355x over previous
import functools

import jax
import jax.numpy as jnp
from jax import lax
from jax.experimental import pallas as pl
from jax.experimental.pallas import tpu as pltpu
from jax.experimental.pallas import tpu_sc as plsc

F32 = jnp.float32
LAT = 128
LN_EPS = 1e-5

NC, NS = 2, 16
NW = NC * NS
CHUNK = 80


def _dot(a, b):
    return jnp.dot(a, b, preferred_element_type=F32)


def _layernorm(y, g, b):
    m = jnp.mean(y, axis=-1, keepdims=True)
    v = jnp.mean((y - m) * (y - m), axis=-1, keepdims=True)
    return (y - m) * lax.rsqrt(v + LN_EPS) * g + b


def _full(shape):
    return pl.BlockSpec(shape, lambda i: (0,) * len(shape))


def _rows(br, w):
    return pl.BlockSpec((br, w), lambda i: (i, 0))


def _mlp_tail(h, w2, b2, w3, b3, w4, b4):
    h = jnp.maximum(_dot(h, w2) + b2, 0.0)
    h = jnp.maximum(_dot(h, w3) + b3, 0.0)
    return _dot(h, w4) + b4


def _encode_body(x_ref, w1, b1, w2, b2, w3, b3, w4, b4, lng, lnb, wa, wb,
                 node_ref, pa_ref, pb_ref):
    h = jnp.maximum(_dot(x_ref[...], w1[...]) + b1[...], 0.0)
    y = _mlp_tail(h, w2[...], b2[...], w3[...], b3[...], w4[...], b4[...])
    y = _layernorm(y, lng[...], lnb[...])
    node_ref[...] = y
    pa_ref[...] = _dot(y, wa[...])
    pb_ref[...] = _dot(y, wb[...])


def _tc_encode_node(x, mlp, wa, wb, br):
    n = x.shape[0]
    k = x.shape[1]
    args = _mlp_args(mlp)
    return pl.pallas_call(
        _encode_body,
        grid=(n // br,),
        in_specs=[_rows(br, k)] + _weight_specs(k) + [_full((LAT, LAT))] * 2,
        out_specs=[_rows(br, LAT)] * 3,
        out_shape=[jax.ShapeDtypeStruct((n, LAT), F32)] * 3,
    )(x, *args, wa, wb)


def _enc_edge_body(x_ref, w1, b1, w2, b2, w3, b3, w4, b4, lng, lnb, out_ref):
    h = jnp.maximum(_dot(x_ref[...], w1[...]) + b1[...], 0.0)
    y = _mlp_tail(h, w2[...], b2[...], w3[...], b3[...], w4[...], b4[...])
    out_ref[...] = _layernorm(y, lng[...], lnb[...])


def _tc_encode_edge(x, mlp, br):
    n, k = x.shape
    args = _mlp_args(mlp)
    return pl.pallas_call(
        _enc_edge_body,
        grid=(n // br,),
        in_specs=[_rows(br, k)] + _weight_specs(k),
        out_specs=_rows(br, LAT),
        out_shape=jax.ShapeDtypeStruct((n, LAT), F32),
    )(x, *args)


def _edge_step_body(g_ref, e_ref, w1c, b1, w2, b2, w3, b3, w4, b4, lng, lnb,
                    u_ref, eo_ref):
    e = e_ref[...]
    h = jnp.maximum(g_ref[...] + _dot(e, w1c[...]) + b1[...], 0.0)
    y = _mlp_tail(h, w2[...], b2[...], w3[...], b3[...], w4[...], b4[...])
    u = _layernorm(y, lng[...], lnb[...])
    u_ref[...] = u
    eo_ref[...] = e + u


def _tc_edge_step(g, edge, w1c, b1, w2, b2, w3, b3, w4, b4, lng, lnb, br):
    n = g.shape[0]
    return pl.pallas_call(
        _edge_step_body,
        grid=(n // br,),
        in_specs=[_rows(br, LAT), _rows(br, LAT), _full((LAT, LAT)),
                  _full((1, LAT)), _full((LAT, LAT)), _full((1, LAT)),
                  _full((LAT, LAT)), _full((1, LAT)), _full((LAT, LAT)),
                  _full((1, LAT)), _full((1, LAT)), _full((1, LAT))],
        out_specs=[_rows(br, LAT)] * 2,
        out_shape=[jax.ShapeDtypeStruct((n, LAT), F32)] * 2,
    )(g, edge, w1c, b1, w2, b2, w3, b3, w4, b4, lng, lnb)


def _node_step_body(has_proj, node_ref, aggp_ref, v1a, v1b, b1, w2, b2, w3,
                    b3, w4, b4, lng, lnb, *rest):
    node = node_ref[...]
    agg = aggp_ref[0] + aggp_ref[1]
    h = jnp.maximum(_dot(node, v1a[...]) + _dot(agg, v1b[...]) + b1[...], 0.0)
    y = _mlp_tail(h, w2[...], b2[...], w3[...], b3[...], w4[...], b4[...])
    y = _layernorm(y, lng[...], lnb[...])
    nn = node + y
    if has_proj:
        wa, wb, out_ref, pa_ref, pb_ref = rest
        out_ref[...] = nn
        pa_ref[...] = _dot(nn, wa[...])
        pb_ref[...] = _dot(nn, wb[...])
    else:
        (out_ref,) = rest
        out_ref[...] = nn


def _tc_node_step(node, aggp, v1a, v1b, b1, w2, b2, w3, b3, w4, b4, lng, lnb,
                  proj, br):
    n = node.shape[0]
    w_specs = [_full((LAT, LAT)), _full((LAT, LAT)), _full((1, LAT)),
               _full((LAT, LAT)), _full((1, LAT)), _full((LAT, LAT)),
               _full((1, LAT)), _full((LAT, LAT)), _full((1, LAT)),
               _full((1, LAT)), _full((1, LAT))]
    agg_spec = pl.BlockSpec((2, br, LAT), lambda i: (0, i, 0))
    args = [node, aggp, v1a, v1b, b1, w2, b2, w3, b3, w4, b4, lng, lnb]
    if proj is not None:
        wa, wb = proj
        return pl.pallas_call(
            functools.partial(_node_step_body, True),
            grid=(n // br,),
            in_specs=[_rows(br, LAT), agg_spec] + w_specs
                     + [_full((LAT, LAT))] * 2,
            out_specs=[_rows(br, LAT)] * 3,
            out_shape=[jax.ShapeDtypeStruct((n, LAT), F32)] * 3,
        )(*args, wa, wb)
    return pl.pallas_call(
        functools.partial(_node_step_body, False),
        grid=(n // br,),
        in_specs=[_rows(br, LAT), agg_spec] + w_specs,
        out_specs=_rows(br, LAT),
        out_shape=jax.ShapeDtypeStruct((n, LAT), F32),
    )(*args)


def _decoder_body(x_ref, w1, b1, w2, b2, w3, b3, w4, b4, out_ref):
    h = jnp.maximum(_dot(x_ref[...], w1[...]) + b1[...], 0.0)
    out_ref[...] = _mlp_tail(h, w2[...], b2[...], w3[...], b3[...], w4[...],
                             b4[...])


def _tc_decoder(x, mlp, br):
    n = x.shape[0]
    out = mlp["layers"][3]["w"].shape[1]
    args = _mlp_args(mlp)
    return pl.pallas_call(
        _decoder_body,
        grid=(n // br,),
        in_specs=[_rows(br, LAT), _full((LAT, LAT)), _full((1, LAT)),
                  _full((LAT, LAT)), _full((1, LAT)), _full((LAT, LAT)),
                  _full((1, LAT)), _full((LAT, out)), _full((1, out))],
        out_specs=_rows(br, out),
        out_shape=jax.ShapeDtypeStruct((n, out), F32),
    )(x, *args)


def _weight_specs(k_in):
    return [_full((k_in, LAT)), _full((1, LAT)), _full((LAT, LAT)),
            _full((1, LAT)), _full((LAT, LAT)), _full((1, LAT)),
            _full((LAT, LAT)), _full((1, LAT)), _full((1, LAT)),
            _full((1, LAT))]


def _mlp_args(mlp):
    out = []
    for l in mlp["layers"]:
        out.append(l["w"])
        out.append(l["b"].reshape(1, -1))
    if "ln" in mlp:
        out.append(mlp["ln"]["g"].reshape(1, -1))
        out.append(mlp["ln"]["b"].reshape(1, -1))
    return out


def sc_gather_sum(pa, pb, s3, r3):
    nw, nchunk, chunk = s3.shape
    per_w = nchunk * chunk
    n_edges = nw * per_w
    mesh = plsc.VectorSubcoreMesh(core_axis_name="c", subcore_axis_name="s")

    @functools.partial(
        pl.kernel, mesh=mesh,
        out_type=jax.ShapeDtypeStruct((n_edges, LAT), F32),
        scratch_types=[
            pltpu.VMEM((nchunk, chunk), jnp.int32),
            pltpu.VMEM((nchunk, chunk), jnp.int32),
            pltpu.VMEM((chunk, LAT), F32),
            pltpu.VMEM((chunk, LAT), F32),
            pltpu.SemaphoreType.DMA,
            pltpu.SemaphoreType.DMA,
        ],
    )
    def k(pa_h, pb_h, s_h, r_h, out_h, sidx, ridx, rows_a, rows_b, sem_a,
          sem_b):
        wid = lax.axis_index("c") * NS + lax.axis_index("s")
        pltpu.sync_copy(s_h.at[wid], sidx)
        pltpu.sync_copy(r_h.at[wid], ridx)

        def body(j, _):
            cp_a = pltpu.async_copy(pa_h.at[sidx.at[j]], rows_a, sem_a)
            cp_b = pltpu.async_copy(pb_h.at[ridx.at[j]], rows_b, sem_b)
            cp_a.wait()
            cp_b.wait()

            def add_body(t, _):
                i = t // (LAT // 16)
                c0 = (t % (LAT // 16)) * 16
                rows_a[i, pl.ds(c0, 16)] = (rows_a[i, pl.ds(c0, 16)]
                                            + rows_b[i, pl.ds(c0, 16)])
                return 0

            lax.fori_loop(0, chunk * (LAT // 16), add_body, 0)
            pltpu.sync_copy(
                rows_a, out_h.at[pl.ds(wid * per_w + j * chunk, chunk)])
            return 0

        lax.fori_loop(0, nchunk, body, 0)

    return k(pa, pb, s3, r3)


def sc_scatter_add(u, r3, zeros, n_pad):
    nw, nchunk, chunk = r3.shape
    per_w = nchunk * chunk
    rows_per_tile = n_pad // NS
    mesh = plsc.VectorSubcoreMesh(core_axis_name="c", subcore_axis_name="s")

    @functools.partial(
        pl.kernel, mesh=mesh,
        out_type=jax.ShapeDtypeStruct((NC, n_pad, LAT), F32),
        scratch_types=[
            pltpu.VMEM((nchunk, chunk), jnp.int32),
            pltpu.VMEM((chunk, LAT), F32),
            pltpu.VMEM_SHARED((n_pad, LAT), F32),
        ],
    )
    def k(u_h, r_h, z_h, out_h, ridx, rows, accum):
        c = lax.axis_index("c")
        s = lax.axis_index("s")
        wid = c * NS + s
        pltpu.sync_copy(r_h.at[wid], ridx)
        pltpu.sync_copy(z_h.at[pl.ds(s * rows_per_tile, rows_per_tile)],
                        accum.at[pl.ds(s * rows_per_tile, rows_per_tile)])
        plsc.subcore_barrier()

        def body(j, _):
            pltpu.sync_copy(
                u_h.at[pl.ds(wid * per_w + j * chunk, chunk)], rows)
            pltpu.sync_copy(rows, accum.at[ridx.at[j]], add=True)
            return 0

        lax.fori_loop(0, nchunk, body, 0)
        plsc.subcore_barrier()
        pltpu.sync_copy(
            accum.at[pl.ds(s * rows_per_tile, rows_per_tile)],
            out_h.at[c, pl.ds(s * rows_per_tile, rows_per_tile)])

    return k(u, r3, zeros)


def kernel(x, edge_index, edge_attr, params):
    n_nodes = x.shape[0]
    n_edges = edge_attr.shape[0]
    per_w = n_edges // NW
    nchunk = per_w // CHUNK

    senders = edge_index[0].astype(jnp.int32)
    receivers = edge_index[1].astype(jnp.int32)
    s3 = senders.reshape(NW, nchunk, CHUNK)
    r3 = receivers.reshape(NW, nchunk, CHUNK)
    n_pad = ((n_nodes + 8 * NS - 1) // (8 * NS)) * (8 * NS)
    zeros = jnp.zeros((n_pad, LAT), F32)

    blocks = params["blocks"]

    def split_w1(blk):
        w1 = blk["edge_mlp"]["layers"][0]["w"]
        return w1[:LAT], w1[LAT:2 * LAT], w1[2 * LAT:]

    wa0, wb0, _ = split_w1(blocks[0])
    node, pa, pb = _tc_encode_node(x, params["node_enc"], wa0, wb0, br=2000)
    edge = _tc_encode_edge(edge_attr, params["edge_enc"], br=2000)

    for k in range(len(blocks)):
        blk = blocks[k]
        em = blk["edge_mlp"]
        nm = blk["node_mlp"]
        _, _, w1c = split_w1(blk)
        g = sc_gather_sum(pa, pb, s3, r3)
        u, edge = _tc_edge_step(
            g, edge, w1c,
            em["layers"][0]["b"].reshape(1, -1),
            em["layers"][1]["w"], em["layers"][1]["b"].reshape(1, -1),
            em["layers"][2]["w"], em["layers"][2]["b"].reshape(1, -1),
            em["layers"][3]["w"], em["layers"][3]["b"].reshape(1, -1),
            em["ln"]["g"].reshape(1, -1), em["ln"]["b"].reshape(1, -1),
            br=2000)
        aggp = sc_scatter_add(u, r3, zeros, n_pad)[:, :n_nodes]
        v1 = nm["layers"][0]["w"]
        proj = None
        if k + 1 < len(blocks):
            wa, wb, _ = split_w1(blocks[k + 1])
            proj = (wa, wb)
        res = _tc_node_step(
            node, aggp, v1[:LAT], v1[LAT:],
            nm["layers"][0]["b"].reshape(1, -1),
            nm["layers"][1]["w"], nm["layers"][1]["b"].reshape(1, -1),
            nm["layers"][2]["w"], nm["layers"][2]["b"].reshape(1, -1),
            nm["layers"][3]["w"], nm["layers"][3]["b"].reshape(1, -1),
            nm["ln"]["g"].reshape(1, -1), nm["ln"]["b"].reshape(1, -1),
            proj, br=2000)
        if proj is not None:
            node, pa, pb = res
        else:
            node = res

    return _tc_decoder(node, params["decoder"], br=2000)


# speedup vs baseline: 3.4497x; 1.3605x over previous
import functools

import jax
import jax.numpy as jnp
from jax import lax
from jax.experimental import pallas as pl
from jax.experimental.pallas import tpu as pltpu
from jax.experimental.pallas import tpu_sc as plsc

F32 = jnp.float32
LAT = 128
HALF = LAT // 2
LN_EPS = 1e-5

NC, NS = 2, 16
NW = NC * NS
CHUNK = 40
GRP = 5


def _dot(a, b):
    return jnp.dot(a, b, preferred_element_type=F32)


def _layernorm(y, g, b):
    m = jnp.mean(y, axis=-1, keepdims=True)
    v = jnp.mean((y - m) * (y - m), axis=-1, keepdims=True)
    return (y - m) * lax.rsqrt(v + LN_EPS) * g + b


def _full(shape):
    return pl.BlockSpec(shape, lambda i: (0,) * len(shape))


def _rows(br, w):
    return pl.BlockSpec((br, w), lambda i: (i, 0))


def _mlp_tail(h, w2, b2, w3, b3, w4, b4):
    h = jnp.maximum(_dot(h, w2) + b2, 0.0)
    h = jnp.maximum(_dot(h, w3) + b3, 0.0)
    return _dot(h, w4) + b4


def _encode_body(x_ref, w1, b1, w2, b2, w3, b3, w4, b4, lng, lnb, wa, wb,
                 node_ref, pa_ref, pb_ref):
    h = jnp.maximum(_dot(x_ref[...], w1[...]) + b1[...], 0.0)
    y = _mlp_tail(h, w2[...], b2[...], w3[...], b3[...], w4[...], b4[...])
    y = _layernorm(y, lng[...], lnb[...])
    node_ref[...] = y
    pa_ref[...] = _dot(y, wa[...])
    pb_ref[...] = _dot(y, wb[...])


def _tc_encode_node(x, mlp, wa, wb, br):
    n = x.shape[0]
    k = x.shape[1]
    args = _mlp_args(mlp)
    return pl.pallas_call(
        _encode_body,
        grid=(n // br,),
        in_specs=[_rows(br, k)] + _weight_specs(k) + [_full((LAT, LAT))] * 2,
        out_specs=[_rows(br, LAT)] * 3,
        out_shape=[jax.ShapeDtypeStruct((n, LAT), F32)] * 3,
    )(x, *args, wa, wb)


def _enc_edge_body(x_ref, w1, b1, w2, b2, w3, b3, w4, b4, lng, lnb, out_ref):
    h = jnp.maximum(_dot(x_ref[...], w1[...]) + b1[...], 0.0)
    y = _mlp_tail(h, w2[...], b2[...], w3[...], b3[...], w4[...], b4[...])
    out_ref[...] = _layernorm(y, lng[...], lnb[...])


def _tc_encode_edge(x, mlp, br):
    n, k = x.shape
    args = _mlp_args(mlp)
    return pl.pallas_call(
        _enc_edge_body,
        grid=(n // br,),
        in_specs=[_rows(br, k)] + _weight_specs(k),
        out_specs=_rows(br, LAT),
        out_shape=jax.ShapeDtypeStruct((n, LAT), F32),
    )(x, *args)


def _edge_step_body(g_ref, e_ref, w1c, b1, w2, b2, w3, b3, w4, b4, lng, lnb,
                    u2_ref, eo_ref):
    e = e_ref[...]
    h = jnp.maximum(g_ref[...] + _dot(e, w1c[...]) + b1[...], 0.0)
    y = _mlp_tail(h, w2[...], b2[...], w3[...], b3[...], w4[...], b4[...])
    u = _layernorm(y, lng[...], lnb[...])
    u2_ref[...] = u
    eo_ref[...] = e + u


def _tc_edge_step(g, edge, w1c, b1, w2, b2, w3, b3, w4, b4, lng, lnb, br):
    n = g.shape[0]
    return pl.pallas_call(
        _edge_step_body,
        grid=(n // br,),
        in_specs=[_rows(br, LAT), _rows(br, LAT), _full((LAT, LAT)),
                  _full((1, LAT)), _full((LAT, LAT)), _full((1, LAT)),
                  _full((LAT, LAT)), _full((1, LAT)), _full((LAT, LAT)),
                  _full((1, LAT)), _full((1, LAT)), _full((1, LAT))],
        out_specs=[_rows(br, LAT)] * 2,
        out_shape=[jax.ShapeDtypeStruct((n, LAT), F32)] * 2,
    )(g, edge, w1c, b1, w2, b2, w3, b3, w4, b4, lng, lnb)


def _node_step_body(has_proj, node_ref, aggp_ref, v1a, v1b, b1, w2,
                    b2, w3, b3, w4, b4, lng, lnb, *rest):
    node = node_ref[...]
    agg = aggp_ref[0] + aggp_ref[1]
    h = jnp.maximum(_dot(node, v1a[...]) + _dot(agg, v1b[...]) + b1[...],
                    0.0)
    y = _mlp_tail(h, w2[...], b2[...], w3[...], b3[...], w4[...], b4[...])
    y = _layernorm(y, lng[...], lnb[...])
    nn = node + y
    if has_proj:
        wa, wb, out_ref, pa_ref, pb_ref = rest
        out_ref[...] = nn
        pa_ref[...] = _dot(nn, wa[...])
        pb_ref[...] = _dot(nn, wb[...])
    else:
        (out_ref,) = rest
        out_ref[...] = nn


def _tc_node_step(node, aggp, v1a, v1b, b1, w2, b2, w3, b3, w4, b4,
                  lng, lnb, proj, br):
    n = node.shape[0]
    w_specs = [_full((LAT, LAT)), _full((LAT, LAT)), _full((1, LAT)),
               _full((LAT, LAT)), _full((1, LAT)), _full((LAT, LAT)),
               _full((1, LAT)), _full((LAT, LAT)), _full((1, LAT)),
               _full((1, LAT)), _full((1, LAT))]
    agg_spec = pl.BlockSpec((2, br, LAT), lambda i: (0, i, 0))
    args = [node, aggp, v1a, v1b, b1, w2, b2, w3, b3, w4, b4, lng, lnb]
    if proj is not None:
        wa, wb = proj
        return pl.pallas_call(
            functools.partial(_node_step_body, True),
            grid=(n // br,),
            in_specs=[_rows(br, LAT), agg_spec] + w_specs
                     + [_full((LAT, LAT))] * 2,
            out_specs=[_rows(br, LAT)] * 3,
            out_shape=[jax.ShapeDtypeStruct((n, LAT), F32)] * 3,
        )(*args, wa, wb)
    return pl.pallas_call(
        functools.partial(_node_step_body, False),
        grid=(n // br,),
        in_specs=[_rows(br, LAT), agg_spec] + w_specs,
        out_specs=_rows(br, LAT),
        out_shape=jax.ShapeDtypeStruct((n, LAT), F32),
    )(*args)


def _decoder_body(x_ref, w1, b1, w2, b2, w3, b3, w4, b4, out_ref):
    h = jnp.maximum(_dot(x_ref[...], w1[...]) + b1[...], 0.0)
    out_ref[...] = _mlp_tail(h, w2[...], b2[...], w3[...], b3[...], w4[...],
                             b4[...])


def _tc_decoder(x, mlp, br):
    n = x.shape[0]
    out = mlp["layers"][3]["w"].shape[1]
    args = _mlp_args(mlp)
    return pl.pallas_call(
        _decoder_body,
        grid=(n // br,),
        in_specs=[_rows(br, LAT), _full((LAT, LAT)), _full((1, LAT)),
                  _full((LAT, LAT)), _full((1, LAT)), _full((LAT, LAT)),
                  _full((1, LAT)), _full((LAT, out)), _full((1, out))],
        out_specs=_rows(br, out),
        out_shape=jax.ShapeDtypeStruct((n, out), F32),
    )(x, *args)


def _weight_specs(k_in):
    return [_full((k_in, LAT)), _full((1, LAT)), _full((LAT, LAT)),
            _full((1, LAT)), _full((LAT, LAT)), _full((1, LAT)),
            _full((LAT, LAT)), _full((1, LAT)), _full((1, LAT)),
            _full((1, LAT))]


def _mlp_args(mlp):
    out = []
    for l in mlp["layers"]:
        out.append(l["w"])
        out.append(l["b"].reshape(1, -1))
    if "ln" in mlp:
        out.append(mlp["ln"]["g"].reshape(1, -1))
        out.append(mlp["ln"]["b"].reshape(1, -1))
    return out


def sc_gather_sum(pa, pb, sr4):
    nw, nbody, nrow, chunk = sr4.shape
    per_w = nbody * 2 * GRP * chunk
    n_edges = nw * per_w
    grows = GRP * chunk
    mesh = plsc.VectorSubcoreMesh(core_axis_name="c", subcore_axis_name="s")

    @functools.partial(
        pl.kernel, mesh=mesh,
        out_type=jax.ShapeDtypeStruct((n_edges, LAT), F32),
        scratch_types=[
            pltpu.VMEM((nrow, chunk), jnp.int32),
            pltpu.VMEM((grows, LAT), F32),
            pltpu.VMEM((grows, LAT), F32),
            pltpu.VMEM((grows, LAT), F32),
            pltpu.VMEM((grows, LAT), F32),
            pltpu.SemaphoreType.DMA,
            pltpu.SemaphoreType.DMA,
            pltpu.SemaphoreType.DMA,
            pltpu.SemaphoreType.DMA,
        ],
    )
    def k(pa_h, pb_h, sr_h, out_h, idxb, a0, b0, a1, b1, sem0,
          sem1, sem_s0, sem_s1):
        wid = lax.axis_index("c") * NS + lax.axis_index("s")
        base = wid * per_w

        def issue(slot, abuf, bbuf, sem):
            cps = []
            for kk in range(GRP):
                dst = pl.ds(kk * chunk, chunk)
                cps.append(pltpu.async_copy(
                    pa_h.at[idxb.at[slot * GRP + kk]], abuf.at[dst], sem))
                cps.append(pltpu.async_copy(
                    pb_h.at[idxb.at[2 * GRP + slot * GRP + kk]],
                    bbuf.at[dst], sem))
            return cps

        def addrows(abuf, bbuf):
            def add4(i, _):
                for rr in range(4):
                    for cc in range(LAT // 16):
                        sl = pl.ds(cc * 16, 16)
                        plsc.addupdate(abuf.at[i * 4 + rr, sl],
                                       bbuf[i * 4 + rr, sl])
                return 0
            lax.fori_loop(0, grows // 4, add4, 0)

        def body(t, _):
            pltpu.sync_copy(sr_h.at[wid, t], idxb)
            cps0 = issue(0, a0, b0, sem0)
            cps1 = issue(1, a1, b1, sem1)
            for cp in cps0:
                cp.wait()
            addrows(a0, b0)
            st0 = pltpu.async_copy(
                a0, out_h.at[pl.ds(base + t * 2 * grows, grows)], sem_s0)
            for cp in cps1:
                cp.wait()
            addrows(a1, b1)
            st1 = pltpu.async_copy(
                a1, out_h.at[pl.ds(base + t * 2 * grows + grows, grows)],
                sem_s1)
            st0.wait()
            st1.wait()
            return 0

        lax.fori_loop(0, nbody, body, 0)

    return k(pa, pb, sr4)


def sc_scatter_add(u, r3, zeros, n_pad):
    nw, nchunk, chunk = r3.shape
    per_w = nchunk * chunk
    rows_per_tile = n_pad // NS
    mesh = plsc.VectorSubcoreMesh(core_axis_name="c", subcore_axis_name="s")

    @functools.partial(
        pl.kernel, mesh=mesh,
        out_type=jax.ShapeDtypeStruct((NC, n_pad, LAT), F32),
        scratch_types=[
            pltpu.VMEM((nchunk, chunk), jnp.int32),
            pltpu.VMEM((chunk, LAT), F32),
            pltpu.VMEM_SHARED((n_pad, LAT), F32),
        ],
    )
    def k(u_h, r_h, z_h, out_h, ridx, rows, accum):
        c = lax.axis_index("c")
        s = lax.axis_index("s")
        wid = c * NS + s
        pltpu.sync_copy(r_h.at[wid], ridx)
        pltpu.sync_copy(z_h.at[pl.ds(s * rows_per_tile, rows_per_tile)],
                        accum.at[pl.ds(s * rows_per_tile, rows_per_tile)])
        plsc.subcore_barrier()

        def body(j, _):
            pltpu.sync_copy(
                u_h.at[pl.ds(wid * per_w + j * chunk, chunk)], rows)
            pltpu.sync_copy(rows, accum.at[ridx.at[j]], add=True)
            return 0

        lax.fori_loop(0, nchunk, body, 0)
        plsc.subcore_barrier()
        pltpu.sync_copy(
            accum.at[pl.ds(s * rows_per_tile, rows_per_tile)],
            out_h.at[c, pl.ds(s * rows_per_tile, rows_per_tile)])

    return k(u, r3, zeros)


def kernel(x, edge_index, edge_attr, params):
    n_nodes = x.shape[0]
    n_edges = edge_attr.shape[0]
    per_w = n_edges // NW
    nchunk = per_w // CHUNK

    senders = edge_index[0].astype(jnp.int32)
    receivers = edge_index[1].astype(jnp.int32)
    s3 = senders.reshape(NW, nchunk, CHUNK)
    r3 = receivers.reshape(NW, nchunk, CHUNK)
    nbody = nchunk // (2 * GRP)
    sr4 = jnp.concatenate(
        [s3.reshape(NW, nbody, 2 * GRP, CHUNK),
         r3.reshape(NW, nbody, 2 * GRP, CHUNK)], axis=2)
    r3sc = receivers.reshape(NW, (n_edges // NW) // 80, 80)
    n_pad = ((n_nodes + 8 * NS - 1) // (8 * NS)) * (8 * NS)
    zeros = jnp.zeros((n_pad, LAT), F32)

    blocks = params["blocks"]

    def split_w1(blk):
        w1 = blk["edge_mlp"]["layers"][0]["w"]
        return w1[:LAT], w1[LAT:2 * LAT], w1[2 * LAT:]

    wa0, wb0, _ = split_w1(blocks[0])
    node, pa, pb = _tc_encode_node(x, params["node_enc"], wa0, wb0, br=2000)
    edge = _tc_encode_edge(edge_attr, params["edge_enc"], br=2000)

    for k in range(len(blocks)):
        blk = blocks[k]
        em = blk["edge_mlp"]
        nm = blk["node_mlp"]
        _, _, w1c = split_w1(blk)
        g = sc_gather_sum(pa, pb, sr4)
        u2, edge = _tc_edge_step(
            g, edge, w1c,
            em["layers"][0]["b"].reshape(1, -1),
            em["layers"][1]["w"], em["layers"][1]["b"].reshape(1, -1),
            em["layers"][2]["w"], em["layers"][2]["b"].reshape(1, -1),
            em["layers"][3]["w"], em["layers"][3]["b"].reshape(1, -1),
            em["ln"]["g"].reshape(1, -1), em["ln"]["b"].reshape(1, -1),
            br=2000)
        aggp = sc_scatter_add(u2, r3sc, zeros, n_pad)[:, :n_nodes]
        v1 = nm["layers"][0]["w"]
        proj = None
        if k + 1 < len(blocks):
            wa, wb, _ = split_w1(blocks[k + 1])
            proj = (wa, wb)
        res = _tc_node_step(
            node, aggp, v1[:LAT], v1[LAT:],
            nm["layers"][0]["b"].reshape(1, -1),
            nm["layers"][1]["w"], nm["layers"][1]["b"].reshape(1, -1),
            nm["layers"][2]["w"], nm["layers"][2]["b"].reshape(1, -1),
            nm["layers"][3]["w"], nm["layers"][3]["b"].reshape(1, -1),
            nm["ln"]["g"].reshape(1, -1), nm["ln"]["b"].reshape(1, -1),
            proj, br=2000)
        if proj is not None:
            node, pa, pb = res
        else:
            node = res

    return _tc_decoder(node, params["decoder"], br=2000)


# speedup vs baseline: 3.7969x; 1.1006x over previous
import functools

import jax
import jax.numpy as jnp
from jax import lax
from jax.experimental import pallas as pl
from jax.experimental.pallas import tpu as pltpu
from jax.experimental.pallas import tpu_sc as plsc

F32 = jnp.float32
LAT = 128
HALF = LAT // 2
LN_EPS = 1e-5

NC, NS = 2, 16
NW = NC * NS
CHUNK = 40
GRP = 5


def _dot(a, b):
    return jnp.dot(a, b, preferred_element_type=F32)


def _layernorm(y, g, b):
    m = jnp.mean(y, axis=-1, keepdims=True)
    v = jnp.mean((y - m) * (y - m), axis=-1, keepdims=True)
    return (y - m) / jnp.sqrt(v + LN_EPS) * g + b


def _full(shape):
    return pl.BlockSpec(shape, lambda i: (0,) * len(shape))


def _rows(br, w):
    return pl.BlockSpec((br, w), lambda i: (i, 0))


def _mlp_tail(h, w2, b2, w3, b3, w4, b4):
    h = jnp.maximum(_dot(h, w2) + b2, 0.0)
    h = jnp.maximum(_dot(h, w3) + b3, 0.0)
    return _dot(h, w4) + b4


def _encode_body(x_ref, w1, b1, w2, b2, w3, b3, w4, b4, lng, lnb, wa, wb,
                 node_ref, pa_ref, pb_ref):
    h = jnp.maximum(_dot(x_ref[...], w1[...]) + b1[...], 0.0)
    y = _mlp_tail(h, w2[...], b2[...], w3[...], b3[...], w4[...], b4[...])
    y = _layernorm(y, lng[...], lnb[...])
    node_ref[...] = y
    pa_ref[...] = _dot(y, wa[...])
    pb_ref[...] = _dot(y, wb[...])


def _tc_encode_node(x, mlp, wa, wb, br):
    n = x.shape[0]
    k = x.shape[1]
    args = _mlp_args(mlp)
    return pl.pallas_call(
        _encode_body,
        grid=(n // br,),
        in_specs=[_rows(br, k)] + _weight_specs(k) + [_full((LAT, LAT))] * 2,
        out_specs=[_rows(br, LAT)] * 3,
        out_shape=[jax.ShapeDtypeStruct((n, LAT), F32)] * 3,
    )(x, *args, wa, wb)


def _enc_edge_body(x_ref, w1, b1, w2, b2, w3, b3, w4, b4, lng, lnb, out_ref):
    h = jnp.maximum(_dot(x_ref[...], w1[...]) + b1[...], 0.0)
    y = _mlp_tail(h, w2[...], b2[...], w3[...], b3[...], w4[...], b4[...])
    out_ref[...] = _layernorm(y, lng[...], lnb[...])


def _tc_encode_edge(x, mlp, br):
    n, k = x.shape
    args = _mlp_args(mlp)
    return pl.pallas_call(
        _enc_edge_body,
        grid=(n // br,),
        in_specs=[_rows(br, k)] + _weight_specs(k),
        out_specs=_rows(br, LAT),
        out_shape=jax.ShapeDtypeStruct((n, LAT), F32),
    )(x, *args)


def _edge_step_body(g_ref, e_ref, w1c, b1, w2, b2, w3, b3, w4, b4, lng, lnb,
                    u2_ref, eo_ref):
    e = e_ref[...]
    h = jnp.maximum(g_ref[...] + _dot(e, w1c[...]) + b1[...], 0.0)
    y = _mlp_tail(h, w2[...], b2[...], w3[...], b3[...], w4[...], b4[...])
    u = _layernorm(y, lng[...], lnb[...])
    u2_ref[...] = u
    eo_ref[...] = e + u


def _tc_edge_step(g, edge, w1c, b1, w2, b2, w3, b3, w4, b4, lng, lnb, br):
    n = g.shape[0]
    return pl.pallas_call(
        _edge_step_body,
        grid=(n // br,),
        in_specs=[_rows(br, LAT), _rows(br, LAT), _full((LAT, LAT)),
                  _full((1, LAT)), _full((LAT, LAT)), _full((1, LAT)),
                  _full((LAT, LAT)), _full((1, LAT)), _full((LAT, LAT)),
                  _full((1, LAT)), _full((1, LAT)), _full((1, LAT))],
        out_specs=[_rows(br, LAT)] * 2,
        out_shape=[jax.ShapeDtypeStruct((n, LAT), F32)] * 2,
    )(g, edge, w1c, b1, w2, b2, w3, b3, w4, b4, lng, lnb)


def _node_step_body(has_proj, node_ref, aggp_ref, v1a, v1b, b1, w2,
                    b2, w3, b3, w4, b4, lng, lnb, *rest):
    node = node_ref[...]
    agg = aggp_ref[0] + aggp_ref[1]
    h = jnp.maximum(_dot(node, v1a[...]) + _dot(agg, v1b[...]) + b1[...],
                    0.0)
    y = _mlp_tail(h, w2[...], b2[...], w3[...], b3[...], w4[...], b4[...])
    y = _layernorm(y, lng[...], lnb[...])
    nn = node + y
    if has_proj:
        wa, wb, out_ref, pa_ref, pb_ref = rest
        out_ref[...] = nn
        pa_ref[...] = _dot(nn, wa[...])
        pb_ref[...] = _dot(nn, wb[...])
    else:
        (out_ref,) = rest
        out_ref[...] = nn


def _tc_node_step(node, aggp, v1a, v1b, b1, w2, b2, w3, b3, w4, b4,
                  lng, lnb, proj, br):
    n = node.shape[0]
    w_specs = [_full((LAT, LAT)), _full((LAT, LAT)), _full((1, LAT)),
               _full((LAT, LAT)), _full((1, LAT)), _full((LAT, LAT)),
               _full((1, LAT)), _full((LAT, LAT)), _full((1, LAT)),
               _full((1, LAT)), _full((1, LAT))]
    agg_spec = pl.BlockSpec((2, br, LAT), lambda i: (0, i, 0))
    args = [node, aggp, v1a, v1b, b1, w2, b2, w3, b3, w4, b4, lng, lnb]
    if proj is not None:
        wa, wb = proj
        return pl.pallas_call(
            functools.partial(_node_step_body, True),
            grid=(n // br,),
            in_specs=[_rows(br, LAT), agg_spec] + w_specs
                     + [_full((LAT, LAT))] * 2,
            out_specs=[_rows(br, LAT)] * 3,
            out_shape=[jax.ShapeDtypeStruct((n, LAT), F32)] * 3,
        )(*args, wa, wb)
    return pl.pallas_call(
        functools.partial(_node_step_body, False),
        grid=(n // br,),
        in_specs=[_rows(br, LAT), agg_spec] + w_specs,
        out_specs=_rows(br, LAT),
        out_shape=jax.ShapeDtypeStruct((n, LAT), F32),
    )(*args)


def _decoder_body(x_ref, w1, b1, w2, b2, w3, b3, w4, b4, out_ref):
    h = jnp.maximum(_dot(x_ref[...], w1[...]) + b1[...], 0.0)
    out_ref[...] = _mlp_tail(h, w2[...], b2[...], w3[...], b3[...], w4[...],
                             b4[...])


def _tc_decoder(x, mlp, br):
    n = x.shape[0]
    out = mlp["layers"][3]["w"].shape[1]
    args = _mlp_args(mlp)
    return pl.pallas_call(
        _decoder_body,
        grid=(n // br,),
        in_specs=[_rows(br, LAT), _full((LAT, LAT)), _full((1, LAT)),
                  _full((LAT, LAT)), _full((1, LAT)), _full((LAT, LAT)),
                  _full((1, LAT)), _full((LAT, out)), _full((1, out))],
        out_specs=_rows(br, out),
        out_shape=jax.ShapeDtypeStruct((n, out), F32),
    )(x, *args)


def _weight_specs(k_in):
    return [_full((k_in, LAT)), _full((1, LAT)), _full((LAT, LAT)),
            _full((1, LAT)), _full((LAT, LAT)), _full((1, LAT)),
            _full((LAT, LAT)), _full((1, LAT)), _full((1, LAT)),
            _full((1, LAT))]


def _mlp_args(mlp):
    out = []
    for l in mlp["layers"]:
        out.append(l["w"])
        out.append(l["b"].reshape(1, -1))
    if "ln" in mlp:
        out.append(mlp["ln"]["g"].reshape(1, -1))
        out.append(mlp["ln"]["b"].reshape(1, -1))
    return out


def sc_gather_sum(pa, pb, sr4):
    nw, nbody, nrow, chunk = sr4.shape
    per_w = nbody * 2 * GRP * chunk
    n_edges = nw * per_w
    grows = GRP * chunk
    mesh = plsc.VectorSubcoreMesh(core_axis_name="c", subcore_axis_name="s")

    @functools.partial(
        pl.kernel, mesh=mesh,
        out_type=jax.ShapeDtypeStruct((n_edges, LAT), F32),
        scratch_types=[
            pltpu.VMEM((nrow, chunk), jnp.int32),
            pltpu.VMEM((grows, LAT), F32),
            pltpu.VMEM((grows, LAT), F32),
            pltpu.VMEM((grows, LAT), F32),
            pltpu.VMEM((grows, LAT), F32),
            pltpu.SemaphoreType.DMA,
            pltpu.SemaphoreType.DMA,
            pltpu.SemaphoreType.DMA,
            pltpu.SemaphoreType.DMA,
        ],
    )
    def k(pa_h, pb_h, sr_h, out_h, idxb, a0, b0, a1, b1, sem0,
          sem1, sem_s0, sem_s1):
        wid = lax.axis_index("c") * NS + lax.axis_index("s")
        base = wid * per_w

        def issue(slot, abuf, bbuf, sem):
            cps = []
            for kk in range(GRP):
                dst = pl.ds(kk * chunk, chunk)
                cps.append(pltpu.async_copy(
                    pa_h.at[idxb.at[slot * GRP + kk]], abuf.at[dst], sem))
                cps.append(pltpu.async_copy(
                    pb_h.at[idxb.at[2 * GRP + slot * GRP + kk]],
                    bbuf.at[dst], sem))
            return cps

        def addrows(abuf, bbuf):
            def add4(i, _):
                for rr in range(4):
                    for cc in range(LAT // 16):
                        sl = pl.ds(cc * 16, 16)
                        plsc.addupdate(abuf.at[i * 4 + rr, sl],
                                       bbuf[i * 4 + rr, sl])
                return 0
            lax.fori_loop(0, grows // 4, add4, 0)

        def body(t, _):
            pltpu.sync_copy(sr_h.at[wid, t], idxb)
            cps0 = issue(0, a0, b0, sem0)
            cps1 = issue(1, a1, b1, sem1)
            for cp in cps0:
                cp.wait()
            addrows(a0, b0)
            st0 = pltpu.async_copy(
                a0, out_h.at[pl.ds(base + t * 2 * grows, grows)], sem_s0)
            for cp in cps1:
                cp.wait()
            addrows(a1, b1)
            st1 = pltpu.async_copy(
                a1, out_h.at[pl.ds(base + t * 2 * grows + grows, grows)],
                sem_s1)
            st0.wait()
            st1.wait()
            return 0

        lax.fori_loop(0, nbody, body, 0)

    return k(pa, pb, sr4)


def sc_scatter_add(u, r3, zeros, n_pad):
    nw, nchunk, chunk = r3.shape
    per_w = nchunk * chunk
    rows_per_tile = n_pad // NS
    npair = (nchunk - 1) // 2
    mesh = plsc.VectorSubcoreMesh(core_axis_name="c", subcore_axis_name="s")

    @functools.partial(
        pl.kernel, mesh=mesh,
        out_type=jax.ShapeDtypeStruct((NC, n_pad, LAT), F32),
        scratch_types=[
            pltpu.VMEM((nchunk, chunk), jnp.int32),
            pltpu.VMEM((chunk, LAT), F32),
            pltpu.VMEM((chunk, LAT), F32),
            pltpu.VMEM_SHARED((n_pad, LAT), F32),
            pltpu.SemaphoreType.DMA,
            pltpu.SemaphoreType.DMA,
        ],
    )
    def k(u_h, r_h, z_h, out_h, ridx, rows0, rows1, accum, sem0, sem1):
        c = lax.axis_index("c")
        s = lax.axis_index("s")
        wid = c * NS + s
        base = wid * per_w
        pltpu.sync_copy(r_h.at[wid], ridx)
        pltpu.sync_copy(z_h.at[pl.ds(s * rows_per_tile, rows_per_tile)],
                        accum.at[pl.ds(s * rows_per_tile, rows_per_tile)])
        plsc.subcore_barrier()

        def load(j, buf, sem):
            return pltpu.async_copy(
                u_h.at[pl.ds(base + j * chunk, chunk)], buf, sem)

        def drain(buf, sem):
            pltpu.make_async_copy(u_h.at[pl.ds(0, chunk)], buf, sem).wait()

        load(0, rows0, sem0)

        def body(t, _):
            load(2 * t + 1, rows1, sem1)
            drain(rows0, sem0)
            pltpu.sync_copy(rows0, accum.at[ridx.at[2 * t]], add=True)

            @pl.when(t < npair - 1)
            def _():
                load(2 * t + 2, rows0, sem0)

            drain(rows1, sem1)
            pltpu.sync_copy(rows1, accum.at[ridx.at[2 * t + 1]], add=True)
            return 0

        lax.fori_loop(0, npair, body, 0)
        load(nchunk - 1, rows0, sem0)
        drain(rows0, sem0)
        pltpu.sync_copy(rows0, accum.at[ridx.at[nchunk - 1]], add=True)
        plsc.subcore_barrier()
        pltpu.sync_copy(
            accum.at[pl.ds(s * rows_per_tile, rows_per_tile)],
            out_h.at[c, pl.ds(s * rows_per_tile, rows_per_tile)])

    return k(u, r3, zeros)


def kernel(x, edge_index, edge_attr, params):
    n_nodes = x.shape[0]
    n_edges = edge_attr.shape[0]
    per_w = n_edges // NW
    nchunk = per_w // CHUNK

    senders = edge_index[0].astype(jnp.int32)
    receivers = edge_index[1].astype(jnp.int32)
    s3 = senders.reshape(NW, nchunk, CHUNK)
    r3 = receivers.reshape(NW, nchunk, CHUNK)
    nbody = nchunk // (2 * GRP)
    sr4 = jnp.concatenate(
        [s3.reshape(NW, nbody, 2 * GRP, CHUNK),
         r3.reshape(NW, nbody, 2 * GRP, CHUNK)], axis=2)
    r3sc = receivers.reshape(NW, (n_edges // NW) // 80, 80)
    n_pad = ((n_nodes + 8 * NS - 1) // (8 * NS)) * (8 * NS)
    zeros = jnp.zeros((n_pad, LAT), F32)

    blocks = params["blocks"]

    def split_w1(blk):
        w1 = blk["edge_mlp"]["layers"][0]["w"]
        return w1[:LAT], w1[LAT:2 * LAT], w1[2 * LAT:]

    wa0, wb0, _ = split_w1(blocks[0])
    node, pa, pb = _tc_encode_node(x, params["node_enc"], wa0, wb0, br=2000)
    edge = _tc_encode_edge(edge_attr, params["edge_enc"], br=2000)

    for k in range(len(blocks)):
        blk = blocks[k]
        em = blk["edge_mlp"]
        nm = blk["node_mlp"]
        _, _, w1c = split_w1(blk)
        g = sc_gather_sum(pa, pb, sr4)
        u2, edge = _tc_edge_step(
            g, edge, w1c,
            em["layers"][0]["b"].reshape(1, -1),
            em["layers"][1]["w"], em["layers"][1]["b"].reshape(1, -1),
            em["layers"][2]["w"], em["layers"][2]["b"].reshape(1, -1),
            em["layers"][3]["w"], em["layers"][3]["b"].reshape(1, -1),
            em["ln"]["g"].reshape(1, -1), em["ln"]["b"].reshape(1, -1),
            br=2000)
        aggp = sc_scatter_add(u2, r3sc, zeros, n_pad)[:, :n_nodes]
        v1 = nm["layers"][0]["w"]
        proj = None
        if k + 1 < len(blocks):
            wa, wb, _ = split_w1(blocks[k + 1])
            proj = (wa, wb)
        res = _tc_node_step(
            node, aggp, v1[:LAT], v1[LAT:],
            nm["layers"][0]["b"].reshape(1, -1),
            nm["layers"][1]["w"], nm["layers"][1]["b"].reshape(1, -1),
            nm["layers"][2]["w"], nm["layers"][2]["b"].reshape(1, -1),
            nm["layers"][3]["w"], nm["layers"][3]["b"].reshape(1, -1),
            nm["ln"]["g"].reshape(1, -1), nm["ln"]["b"].reshape(1, -1),
            proj, br=2000)
        if proj is not None:
            node, pa, pb = res
        else:
            node = res

    return _tc_decoder(node, params["decoder"], br=2000)
